# baseline (speedup 1.0000x reference)
import functools

import jax
import jax.numpy as jnp
from jax import lax
from jax.experimental import pallas as pl
from jax.experimental.pallas import tpu as pltpu
from jax.experimental.pallas import tpu_sc as plsc

VOCAB = 1000
DIM = 128
TOKENS = 1024 * 50
NW = 32
BPW = TOKENS // NW
CHUNK = 40
NCHUNKS = BPW // CHUNK


def _logits_table(embed_table, head_w, head_b):
    def mm(e_ref, w_ref, b_ref, o_ref):
        o_ref[...] = lax.dot_general(
            e_ref[...], w_ref[...],
            dimension_numbers=(((1,), (1,)), ((), ())),
            preferred_element_type=jnp.float32,
        ) + b_ref[...]

    return pl.pallas_call(
        mm,
        out_shape=jax.ShapeDtypeStruct((VOCAB, VOCAB), jnp.float32),
    )(embed_table, head_w, head_b.reshape(1, VOCAB))


def _gather_rows(table, idx):
    mesh = plsc.VectorSubcoreMesh(core_axis_name="c", subcore_axis_name="s")

    @functools.partial(
        pl.kernel,
        mesh=mesh,
        out_type=jax.ShapeDtypeStruct((TOKENS, VOCAB), jnp.float32),
        scratch_types=[
            pltpu.VMEM((BPW,), jnp.int32),
            pltpu.VMEM((CHUNK, VOCAB), jnp.float32),
            pltpu.SemaphoreType.DMA,
        ],
        compiler_params=pltpu.CompilerParams(use_tc_tiling_on_sc=False),
    )
    def k(table_hbm, idx_hbm, out_hbm, idx_v, buf, sem):
        wid = lax.axis_index("s") * 2 + lax.axis_index("c")
        base = wid * BPW
        pltpu.sync_copy(idx_hbm.at[pl.ds(base, BPW)], idx_v)

        def body(j, carry):
            pltpu.async_copy(
                table_hbm.at[idx_v.at[pl.ds(j * CHUNK, CHUNK)]], buf, sem
            ).wait()
            pltpu.sync_copy(buf, out_hbm.at[pl.ds(base + j * CHUNK, CHUNK)])
            return carry

        lax.fori_loop(0, NCHUNKS, body, 0)

    return k(table, idx)


def kernel(input_ids, embed_table, head_w, head_b):
    table = _logits_table(embed_table, head_w, head_b)
    idx = input_ids.reshape(TOKENS).astype(jnp.int32)
    flat = _gather_rows(table, idx)
    return flat.reshape(input_ids.shape[0], input_ids.shape[1], VOCAB)


# speedup vs baseline: 1.0359x; 1.0359x over previous
import functools

import jax
import jax.numpy as jnp
from jax import lax
from jax.experimental import pallas as pl
from jax.experimental.pallas import tpu as pltpu
from jax.experimental.pallas import tpu_sc as plsc

VOCAB = 1000
DIM = 128
TOKENS = 1024 * 50
NW = 32
BPW = TOKENS // NW
CHUNK = 40
NCHUNKS = BPW // CHUNK


def _logits_table(embed_table, head_w, head_b):
    def mm(e_ref, w_ref, b_ref, o_ref):
        o_ref[...] = lax.dot_general(
            e_ref[...], w_ref[...],
            dimension_numbers=(((1,), (1,)), ((), ())),
            preferred_element_type=jnp.float32,
        ) + b_ref[...]

    return pl.pallas_call(
        mm,
        out_shape=jax.ShapeDtypeStruct((VOCAB, VOCAB), jnp.float32),
    )(embed_table, head_w, head_b.reshape(1, VOCAB))


def _gather_rows(table, idx):
    mesh = plsc.VectorSubcoreMesh(core_axis_name="c", subcore_axis_name="s")

    @functools.partial(
        pl.kernel,
        mesh=mesh,
        out_type=jax.ShapeDtypeStruct((TOKENS, VOCAB), jnp.float32),
        scratch_types=[
            pltpu.VMEM((BPW,), jnp.int32),
            pltpu.VMEM((CHUNK, VOCAB), jnp.float32),
            pltpu.VMEM((CHUNK, VOCAB), jnp.float32),
            pltpu.SemaphoreType.DMA,
            pltpu.SemaphoreType.DMA,
        ],
        compiler_params=pltpu.CompilerParams(use_tc_tiling_on_sc=False),
    )
    def k(table_hbm, idx_hbm, out_hbm, idx_v, buf0, buf1, sem0, sem1):
        wid = lax.axis_index("s") * 2 + lax.axis_index("c")
        base = wid * BPW
        pltpu.sync_copy(idx_hbm.at[pl.ds(base, BPW)], idx_v)

        def start(j, buf, sem):
            pltpu.async_copy(
                table_hbm.at[idx_v.at[pl.ds(j * CHUNK, CHUNK)]], buf, sem
            )

        def drain_and_store(j, buf, sem):
            pltpu.make_async_copy(
                table_hbm.at[idx_v.at[pl.ds(j * CHUNK, CHUNK)]], buf, sem
            ).wait()
            pltpu.sync_copy(buf, out_hbm.at[pl.ds(base + j * CHUNK, CHUNK)])

        start(0, buf0, sem0)

        def body(g, carry):
            j0 = 2 * g
            start(j0 + 1, buf1, sem1)
            drain_and_store(j0, buf0, sem0)

            @pl.when(g + 1 < NCHUNKS // 2)
            def _():
                start(j0 + 2, buf0, sem0)

            drain_and_store(j0 + 1, buf1, sem1)
            return carry

        lax.fori_loop(0, NCHUNKS // 2, body, 0)

    return k(table, idx)


def kernel(input_ids, embed_table, head_w, head_b):
    table = _logits_table(embed_table, head_w, head_b)
    idx = input_ids.reshape(TOKENS).astype(jnp.int32)
    flat = _gather_rows(table, idx)
    return flat.reshape(input_ids.shape[0], input_ids.shape[1], VOCAB)


# speedup vs baseline: 1.1374x; 1.0979x over previous
import functools

import jax
import jax.numpy as jnp
from jax import lax
from jax.experimental import pallas as pl
from jax.experimental.pallas import tpu as pltpu
from jax.experimental.pallas import tpu_sc as plsc

VOCAB = 1000
DIM = 128
B = 1024
L = 50
LPAD = 56
TOKENS = B * LPAD
NW = 32
BPW = TOKENS // NW
CHUNK = 224
NCHUNKS = BPW // CHUNK
BB = 32


def _gather_rows(table, idx):
    mesh = plsc.VectorSubcoreMesh(core_axis_name="c", subcore_axis_name="s")

    @functools.partial(
        pl.kernel,
        mesh=mesh,
        out_type=jax.ShapeDtypeStruct((TOKENS, DIM), jnp.float32),
        scratch_types=[
            pltpu.VMEM((BPW,), jnp.int32),
            pltpu.VMEM((CHUNK, DIM), jnp.float32),
            pltpu.VMEM((CHUNK, DIM), jnp.float32),
            pltpu.SemaphoreType.DMA,
            pltpu.SemaphoreType.DMA,
        ],
    )
    def k(table_hbm, idx_hbm, out_hbm, idx_v, buf0, buf1, sem0, sem1):
        wid = lax.axis_index("s") * 2 + lax.axis_index("c")
        base = wid * BPW
        pltpu.sync_copy(idx_hbm.at[pl.ds(base, BPW)], idx_v)

        def start(j, buf, sem):
            pltpu.async_copy(
                table_hbm.at[idx_v.at[pl.ds(j * CHUNK, CHUNK)]], buf, sem
            )

        def drain_and_store(j, buf, sem):
            pltpu.make_async_copy(
                table_hbm.at[idx_v.at[pl.ds(j * CHUNK, CHUNK)]], buf, sem
            ).wait()
            pltpu.sync_copy(buf, out_hbm.at[pl.ds(base + j * CHUNK, CHUNK)])

        start(0, buf0, sem0)

        def body(g, carry):
            j0 = 2 * g
            start(j0 + 1, buf1, sem1)
            drain_and_store(j0, buf0, sem0)

            @pl.when(g + 1 < NCHUNKS // 2)
            def _():
                start(j0 + 2, buf0, sem0)

            drain_and_store(j0 + 1, buf1, sem1)
            return carry

        lax.fori_loop(0, NCHUNKS // 2, body, 0)

    return k(table, idx)


def _head_matmul(h3, w_t, bias):
    def mm(h_ref, w_ref, b_ref, o_ref):
        h = h_ref[...][:, :L, :].reshape(BB * L, DIM).astype(jnp.bfloat16)
        w = w_ref[...].astype(jnp.bfloat16)
        acc = jnp.dot(h, w, preferred_element_type=jnp.float32)
        o_ref[...] = acc.reshape(BB, L, VOCAB) + b_ref[...]

    return pl.pallas_call(
        mm,
        grid=(B // BB,),
        in_specs=[
            pl.BlockSpec((BB, LPAD, DIM), lambda i: (i, 0, 0)),
            pl.BlockSpec((DIM, VOCAB), lambda i: (0, 0)),
            pl.BlockSpec((1, VOCAB), lambda i: (0, 0)),
        ],
        out_specs=pl.BlockSpec((BB, L, VOCAB), lambda i: (i, 0, 0)),
        out_shape=jax.ShapeDtypeStruct((B, L, VOCAB), jnp.float32),
    )(h3, w_t, bias)


def kernel(input_ids, embed_table, head_w, head_b):
    ids = input_ids.astype(jnp.int32)
    idx = jnp.pad(ids, ((0, 0), (0, LPAD - L))).reshape(TOKENS)
    h = _gather_rows(embed_table, idx)
    h3 = h.reshape(B, LPAD, DIM)
    return _head_matmul(h3, head_w.T, head_b.reshape(1, VOCAB))


# speedup vs baseline: 5.2762x; 4.6389x over previous
import functools

import jax
import jax.numpy as jnp
from jax import lax
from jax.experimental import pallas as pl
from jax.experimental.pallas import tpu as pltpu
from jax.experimental.pallas import tpu_sc as plsc

VOCAB = 1000
DIM = 128
B = 1024
L = 50
TOKENS = B * L
NW = 32
BPW = TOKENS // NW
CHUNK = 80
NCHUNKS = BPW // CHUNK
NBUF = 4


def _gather_rows(table, idx):
    mesh = plsc.VectorSubcoreMesh(core_axis_name="c", subcore_axis_name="s")

    @functools.partial(
        pl.kernel,
        mesh=mesh,
        out_type=jax.ShapeDtypeStruct((TOKENS, DIM), jnp.float32),
        scratch_types=[
            pltpu.VMEM((BPW,), jnp.int32),
        ]
        + [pltpu.VMEM((CHUNK, DIM), jnp.float32) for _ in range(NBUF)]
        + [pltpu.SemaphoreType.DMA for _ in range(NBUF)],
    )
    def k(table_hbm, idx_hbm, out_hbm, idx_v, *bufs_sems):
        bufs = bufs_sems[:NBUF]
        sems = bufs_sems[NBUF:]
        wid = lax.axis_index("s") * 2 + lax.axis_index("c")
        base = wid * BPW
        pltpu.sync_copy(idx_hbm.at[pl.ds(base, BPW)], idx_v)

        def start(j, q):
            pltpu.async_copy(
                table_hbm.at[idx_v.at[pl.ds(j * CHUNK, CHUNK)]],
                bufs[q], sems[q],
            )

        def drain_and_store(j, q):
            pltpu.make_async_copy(
                table_hbm.at[idx_v.at[pl.ds(j * CHUNK, CHUNK)]],
                bufs[q], sems[q],
            ).wait()
            pltpu.sync_copy(bufs[q], out_hbm.at[pl.ds(base + j * CHUNK, CHUNK)])

        for q in range(NBUF):
            start(q, q)

        def body(g, carry):
            j0 = g * NBUF
            for q in range(NBUF):
                drain_and_store(j0 + q, q)

                @pl.when(j0 + q + NBUF < NCHUNKS)
                def _():
                    start(j0 + q + NBUF, q)

            return carry

        lax.fori_loop(0, NCHUNKS // NBUF, body, 0)

    return k(table, idx)


def _head_matmul(h3, w, b2):
    def mm(h_ref, w_ref, b_ref, o_ref):
        hl = h_ref[...].reshape(B, DIM).astype(jnp.bfloat16)
        wv = w_ref[...].astype(jnp.bfloat16)
        acc = lax.dot_general(
            wv, hl,
            dimension_numbers=(((1,), (1,)), ((), ())),
            preferred_element_type=jnp.float32,
        )
        o_ref[...] = (acc + b_ref[...]).reshape(1, VOCAB, B)

    return pl.pallas_call(
        mm,
        grid=(L,),
        in_specs=[
            pl.BlockSpec((1, B, DIM), lambda l: (l, 0, 0)),
            pl.BlockSpec((VOCAB, DIM), lambda l: (0, 0)),
            pl.BlockSpec((VOCAB, 1), lambda l: (0, 0)),
        ],
        out_specs=pl.BlockSpec((1, VOCAB, B), lambda l: (l, 0, 0)),
        out_shape=jax.ShapeDtypeStruct((L, VOCAB, B), jnp.float32),
    )(h3, w, b2)


def kernel(input_ids, embed_table, head_w, head_b):
    idx = input_ids.astype(jnp.int32).T.reshape(TOKENS)
    h = _gather_rows(embed_table, idx)
    h3 = h.reshape(L, B, DIM)
    out_t = _head_matmul(h3, head_w, head_b.reshape(VOCAB, 1))
    return jnp.transpose(out_t, (2, 0, 1))


# speedup vs baseline: 5.7352x; 1.0870x over previous
import functools

import jax
import jax.numpy as jnp
from jax import lax
from jax.experimental import pallas as pl
from jax.experimental.pallas import tpu as pltpu
from jax.experimental.pallas import tpu_sc as plsc

VOCAB = 1000
DIM = 128
B = 1024
L = 50
TOKENS = B * L
NW = 32
BPW = TOKENS // NW
CHUNK = 80
CPS = 5
SUPER = CHUNK * CPS
NSUPER = BPW // SUPER
LB = 2


def _gather_rows(table, idx):
    mesh = plsc.VectorSubcoreMesh(core_axis_name="c", subcore_axis_name="s")

    @functools.partial(
        pl.kernel,
        mesh=mesh,
        out_type=jax.ShapeDtypeStruct((TOKENS, DIM), jnp.float32),
        scratch_types=[
            pltpu.VMEM((BPW,), jnp.int32),
            pltpu.VMEM((SUPER, DIM), jnp.float32),
            pltpu.VMEM((SUPER, DIM), jnp.float32),
            pltpu.SemaphoreType.DMA,
            pltpu.SemaphoreType.DMA,
            pltpu.SemaphoreType.DMA,
            pltpu.SemaphoreType.DMA,
        ],
    )
    def k(table_hbm, idx_hbm, out_hbm, idx_v, buf0, buf1, sg0, sg1, sw0, sw1):
        bufs = (buf0, buf1)
        sgs = (sg0, sg1)
        sws = (sw0, sw1)
        wid = lax.axis_index("s") * 2 + lax.axis_index("c")
        base = wid * BPW
        pltpu.sync_copy(idx_hbm.at[pl.ds(base, BPW)], idx_v)

        def fire_gathers(s, q):
            for c in range(CPS):
                pltpu.async_copy(
                    table_hbm.at[idx_v.at[pl.ds(s * SUPER + c * CHUNK, CHUNK)]],
                    bufs[q].at[pl.ds(c * CHUNK, CHUNK)], sgs[q],
                )

        def drain_gathers(s, q):
            for c in range(CPS):
                pltpu.make_async_copy(
                    table_hbm.at[idx_v.at[pl.ds(s * SUPER + c * CHUNK, CHUNK)]],
                    bufs[q].at[pl.ds(c * CHUNK, CHUNK)], sgs[q],
                ).wait()

        def write(s, q):
            return pltpu.async_copy(
                bufs[q], out_hbm.at[pl.ds(base + s * SUPER, SUPER)], sws[q]
            )

        def wait_write(s, q):
            pltpu.make_async_copy(
                bufs[q], out_hbm.at[pl.ds(base + s * SUPER, SUPER)], sws[q]
            ).wait()

        fire_gathers(0, 0)
        for s in range(NSUPER):
            q = s % 2
            drain_gathers(s, q)
            if s + 1 < NSUPER:
                if s >= 1:
                    wait_write(s - 1, 1 - q)
                fire_gathers(s + 1, 1 - q)
            write(s, q)
        wait_write(NSUPER - 2, 0 if (NSUPER - 2) % 2 == 0 else 1)
        wait_write(NSUPER - 1, 0 if (NSUPER - 1) % 2 == 0 else 1)

    return k(table, idx)


def _head_matmul(h3, w, b2):
    def mm(h_ref, w_ref, b_ref, o_ref):
        wv = w_ref[...].astype(jnp.bfloat16)
        bv = b_ref[...]
        for i in range(LB):
            hl = h_ref[i].reshape(B, DIM).astype(jnp.bfloat16)
            acc = lax.dot_general(
                wv, hl,
                dimension_numbers=(((1,), (1,)), ((), ())),
                preferred_element_type=jnp.float32,
            )
            o_ref[i] = acc + bv

    return pl.pallas_call(
        mm,
        grid=(L // LB,),
        in_specs=[
            pl.BlockSpec((LB, B, DIM), lambda l: (l, 0, 0)),
            pl.BlockSpec((VOCAB, DIM), lambda l: (0, 0)),
            pl.BlockSpec((VOCAB, 1), lambda l: (0, 0)),
        ],
        out_specs=pl.BlockSpec((LB, VOCAB, B), lambda l: (l, 0, 0)),
        out_shape=jax.ShapeDtypeStruct((L, VOCAB, B), jnp.float32),
    )(h3, w, b2)


def kernel(input_ids, embed_table, head_w, head_b):
    idx = input_ids.astype(jnp.int32).T.reshape(TOKENS)
    h = _gather_rows(embed_table, idx)
    h3 = h.reshape(L, B, DIM)
    out_t = _head_matmul(h3, head_w, head_b.reshape(VOCAB, 1))
    return jnp.transpose(out_t, (2, 0, 1))


# speedup vs baseline: 6.5899x; 1.1490x over previous
import functools

import jax
import jax.numpy as jnp
from jax import lax
from jax.experimental import pallas as pl
from jax.experimental.pallas import tpu as pltpu
from jax.experimental.pallas import tpu_sc as plsc

VOCAB = 1000
DIM = 128
B = 1024
L = 50
TOKENS = B * L
NW = 32
BPW = TOKENS // NW
CHUNK = 80
CPS = 5
SUPER = CHUNK * CPS
NSUPER = BPW // SUPER
LB = 2


def _gather_rows(table, idx):
    mesh = plsc.VectorSubcoreMesh(core_axis_name="c", subcore_axis_name="s")

    @functools.partial(
        pl.kernel,
        mesh=mesh,
        out_type=jax.ShapeDtypeStruct((TOKENS, DIM), jnp.float32),
        scratch_types=[
            pltpu.VMEM((BPW,), jnp.int32),
            pltpu.VMEM((SUPER, DIM), jnp.float32),
            pltpu.VMEM((SUPER, DIM), jnp.float32),
            pltpu.VMEM_SHARED((VOCAB, DIM), jnp.float32),
            pltpu.SemaphoreType.DMA,
            pltpu.SemaphoreType.DMA,
            pltpu.SemaphoreType.DMA,
            pltpu.SemaphoreType.DMA,
        ],
    )
    def k(table_hbm, idx_hbm, out_hbm, idx_v, buf0, buf1, tbl_s,
          sg0, sg1, sw0, sw1):
        bufs = (buf0, buf1)
        sgs = (sg0, sg1)
        sws = (sw0, sw1)
        sid = lax.axis_index("s")
        wid = sid * 2 + lax.axis_index("c")
        base = wid * BPW

        @pl.when(sid == 0)
        def _():
            pltpu.sync_copy(table_hbm, tbl_s)

        pltpu.sync_copy(idx_hbm.at[pl.ds(base, BPW)], idx_v)
        plsc.subcore_barrier()

        def fire_gathers(s, q):
            for c in range(CPS):
                pltpu.async_copy(
                    tbl_s.at[idx_v.at[pl.ds(s * SUPER + c * CHUNK, CHUNK)]],
                    bufs[q].at[pl.ds(c * CHUNK, CHUNK)], sgs[q],
                )

        def drain_gathers(s, q):
            for c in range(CPS):
                pltpu.make_async_copy(
                    tbl_s.at[idx_v.at[pl.ds(s * SUPER + c * CHUNK, CHUNK)]],
                    bufs[q].at[pl.ds(c * CHUNK, CHUNK)], sgs[q],
                ).wait()

        def write(s, q):
            return pltpu.async_copy(
                bufs[q], out_hbm.at[pl.ds(base + s * SUPER, SUPER)], sws[q]
            )

        def wait_write(s, q):
            pltpu.make_async_copy(
                bufs[q], out_hbm.at[pl.ds(base + s * SUPER, SUPER)], sws[q]
            ).wait()

        fire_gathers(0, 0)
        for s in range(NSUPER):
            q = s % 2
            drain_gathers(s, q)
            if s + 1 < NSUPER:
                if s >= 1:
                    wait_write(s - 1, 1 - q)
                fire_gathers(s + 1, 1 - q)
            write(s, q)
        wait_write(NSUPER - 2, 0 if (NSUPER - 2) % 2 == 0 else 1)
        wait_write(NSUPER - 1, 0 if (NSUPER - 1) % 2 == 0 else 1)

    return k(table, idx)


def _head_matmul(h3, w, b2):
    def mm(h_ref, w_ref, b_ref, o_ref):
        wv = w_ref[...].astype(jnp.bfloat16)
        bv = b_ref[...]
        for i in range(LB):
            hl = h_ref[i].reshape(B, DIM).astype(jnp.bfloat16)
            acc = lax.dot_general(
                wv, hl,
                dimension_numbers=(((1,), (1,)), ((), ())),
                preferred_element_type=jnp.float32,
            )
            o_ref[i] = acc + bv

    return pl.pallas_call(
        mm,
        grid=(L // LB,),
        in_specs=[
            pl.BlockSpec((LB, B, DIM), lambda l: (l, 0, 0)),
            pl.BlockSpec((VOCAB, DIM), lambda l: (0, 0)),
            pl.BlockSpec((VOCAB, 1), lambda l: (0, 0)),
        ],
        out_specs=pl.BlockSpec((LB, VOCAB, B), lambda l: (l, 0, 0)),
        out_shape=jax.ShapeDtypeStruct((L, VOCAB, B), jnp.float32),
    )(h3, w, b2)


def kernel(input_ids, embed_table, head_w, head_b):
    idx = input_ids.astype(jnp.int32).T.reshape(TOKENS)
    h = _gather_rows(embed_table, idx)
    h3 = h.reshape(L, B, DIM)
    out_t = _head_matmul(h3, head_w, head_b.reshape(VOCAB, 1))
    return jnp.transpose(out_t, (2, 0, 1))
